# baseline (speedup 1.0000x reference)
import functools

import jax
import jax.numpy as jnp
from jax.experimental import pallas as pl
from jax.experimental.pallas import tpu as pltpu

_LANES = 128
_S = 16
_PACK = _LANES // _S


def _masked_linear_kernel(x_ref, m_ref, w_ref, b_ref, e_ref, o_ref):
    x = x_ref[...]
    y = jnp.dot(x, w_ref[...], preferred_element_type=jnp.float32)
    y = y + b_ref[...]
    mf = m_ref[...].astype(jnp.float32)
    me = jnp.dot(mf, e_ref[...], preferred_element_type=jnp.float32)
    o_ref[...] = y * me


@jax.jit
def kernel(stacks, mask, W, bias):
    b, n, _, s = stacks.shape
    h = W.shape[0]
    total = b * n * n * s
    m_rows = total // _LANES

    x = stacks.reshape(m_rows, _LANES)
    m2 = mask.reshape(m_rows, _PACK)

    w_big = jnp.kron(jnp.eye(_PACK, dtype=W.dtype), W.T)
    bias_big = jnp.tile(bias, _PACK).reshape(1, _LANES)
    expand = (jnp.arange(_LANES) // _S == jnp.arange(_PACK)[:, None]).astype(
        jnp.float32
    )

    block_rows = 4096
    grid = (m_rows // block_rows,)

    out = pl.pallas_call(
        _masked_linear_kernel,
        grid=grid,
        in_specs=[
            pl.BlockSpec((block_rows, _LANES), lambda i: (i, 0)),
            pl.BlockSpec((block_rows, _PACK), lambda i: (i, 0)),
            pl.BlockSpec((_LANES, _LANES), lambda i: (0, 0)),
            pl.BlockSpec((1, _LANES), lambda i: (0, 0)),
            pl.BlockSpec((_PACK, _LANES), lambda i: (0, 0)),
        ],
        out_specs=pl.BlockSpec((block_rows, _LANES), lambda i: (i, 0)),
        out_shape=jax.ShapeDtypeStruct((m_rows, _LANES), jnp.float32),
        compiler_params=pltpu.CompilerParams(
            dimension_semantics=("arbitrary",),
        ),
    )(x, m2, w_big, bias_big, expand)

    return out.reshape(b, n, n, h)


# speedup vs baseline: 11.5510x; 11.5510x over previous
import jax
import jax.numpy as jnp
from jax.experimental import pallas as pl
from jax.experimental.pallas import tpu as pltpu

_S = 16
_PACK = 8
_BLOCK_I = 32
_N_LANES = 512


def _masked_linear_kernel(x_ref, m_ref, a_ref, b_ref, o_ref):
    a = a_ref[...]
    bcol = b_ref[...][:, 0:1]
    mf = m_ref[...].astype(jnp.float32)
    x = x_ref[...]
    for k in range(_BLOCK_I // _PACK):
        xk = x[_PACK * k : _PACK * (k + 1)].reshape(_PACK * _S, _N_LANES)
        y = jnp.dot(a, xk, preferred_element_type=jnp.float32) + bcol
        me = jnp.repeat(mf[_PACK * k : _PACK * k + _PACK, :], _S, axis=0)
        o_ref[_PACK * k : _PACK * (k + 1)] = (y * me).reshape(
            _PACK, _S, _N_LANES
        )


@jax.jit
def kernel(stacks, mask, W, bias):
    b, n, _, s = stacks.shape
    h = W.shape[0]
    rows = b * n

    xt = jnp.transpose(stacks, (0, 1, 3, 2)).reshape(rows, s, n)
    m2 = mask.reshape(rows, n)

    a_big = jnp.kron(jnp.eye(_PACK, dtype=W.dtype), W)
    b_big = jnp.tile(jnp.tile(bias, _PACK)[:, None], (1, _PACK * h))

    grid = (rows // _BLOCK_I,)

    out = pl.pallas_call(
        _masked_linear_kernel,
        grid=grid,
        in_specs=[
            pl.BlockSpec((_BLOCK_I, s, n), lambda i: (i, 0, 0)),
            pl.BlockSpec((_BLOCK_I, n), lambda i: (i, 0)),
            pl.BlockSpec((_PACK * h, _PACK * s), lambda i: (0, 0)),
            pl.BlockSpec((_PACK * h, _PACK * h), lambda i: (0, 0)),
        ],
        out_specs=pl.BlockSpec((_BLOCK_I, h, n), lambda i: (i, 0, 0)),
        out_shape=jax.ShapeDtypeStruct((rows, h, n), jnp.float32),
        compiler_params=pltpu.CompilerParams(
            dimension_semantics=("arbitrary",),
        ),
    )(xt, m2, a_big, b_big)

    return jnp.transpose(out.reshape(b, n, h, n), (0, 1, 3, 2))


# speedup vs baseline: 15.7473x; 1.3633x over previous
import jax
import jax.numpy as jnp
from jax.experimental import pallas as pl
from jax.experimental.pallas import tpu as pltpu

_S = 16
_PACK = 8
_BLOCK_I = 64
_N_LANES = 512


def _masked_linear_kernel(x_ref, m_ref, a_ref, b_ref, o_ref):
    a = a_ref[...]
    bcol = b_ref[...][:, 0:1]
    mf = m_ref[...].astype(jnp.float32)
    x = x_ref[...]
    for k in range(_BLOCK_I // _PACK):
        xk = x[_PACK * k : _PACK * (k + 1)].reshape(_PACK * _S, _N_LANES)
        y = jnp.dot(a, xk, preferred_element_type=jnp.float32) + bcol
        me = jnp.repeat(mf[_PACK * k : _PACK * k + _PACK, :], _S, axis=0)
        o_ref[_PACK * k : _PACK * (k + 1)] = (y * me).reshape(
            _PACK, _S, _N_LANES
        )


@jax.jit
def kernel(stacks, mask, W, bias):
    b, n, _, s = stacks.shape
    h = W.shape[0]
    rows = b * n

    xt = jnp.transpose(stacks, (0, 1, 3, 2)).reshape(rows, s, n)
    m2 = mask.reshape(rows, n)

    a_big = jnp.kron(jnp.eye(_PACK, dtype=W.dtype), W)
    b_big = jnp.tile(jnp.tile(bias, _PACK)[:, None], (1, _PACK * h))

    grid = (rows // _BLOCK_I,)

    out = pl.pallas_call(
        _masked_linear_kernel,
        grid=grid,
        in_specs=[
            pl.BlockSpec((_BLOCK_I, s, n), lambda i: (i, 0, 0)),
            pl.BlockSpec((_BLOCK_I, n), lambda i: (i, 0)),
            pl.BlockSpec((_PACK * h, _PACK * s), lambda i: (0, 0)),
            pl.BlockSpec((_PACK * h, _PACK * h), lambda i: (0, 0)),
        ],
        out_specs=pl.BlockSpec((_BLOCK_I, h, n), lambda i: (i, 0, 0)),
        out_shape=jax.ShapeDtypeStruct((rows, h, n), jnp.float32),
        compiler_params=pltpu.CompilerParams(
            dimension_semantics=("parallel",),
        ),
    )(xt, m2, a_big, b_big)

    return jnp.transpose(out.reshape(b, n, h, n), (0, 1, 3, 2))


# speedup vs baseline: 17.7910x; 1.1298x over previous
import jax
import jax.numpy as jnp
from jax.experimental import pallas as pl
from jax.experimental.pallas import tpu as pltpu

_S = 16
_PACK = 8
_BLOCK_I = 128
_N_LANES = 512


def _masked_linear_kernel(x_ref, m_ref, a_ref, b_ref, o_ref):
    a = a_ref[...]
    bcol = b_ref[...][:, 0:1]
    mf = m_ref[...].astype(jnp.float32)
    x = x_ref[...]
    for k in range(_BLOCK_I // _PACK):
        xk = x[_PACK * k : _PACK * (k + 1)].reshape(_PACK * _S, _N_LANES)
        y = jnp.dot(a, xk, preferred_element_type=jnp.float32) + bcol
        me = jnp.repeat(mf[_PACK * k : _PACK * k + _PACK, :], _S, axis=0)
        o_ref[_PACK * k : _PACK * (k + 1)] = (y * me).reshape(
            _PACK, _S, _N_LANES
        )


@jax.jit
def kernel(stacks, mask, W, bias):
    b, n, _, s = stacks.shape
    h = W.shape[0]
    rows = b * n

    xt = jnp.transpose(stacks, (0, 1, 3, 2)).reshape(rows, s, n)
    m2 = mask.reshape(rows, n)

    a_big = jnp.kron(jnp.eye(_PACK, dtype=W.dtype), W)
    b_big = jnp.tile(jnp.tile(bias, _PACK)[:, None], (1, _PACK * h))

    grid = (rows // _BLOCK_I,)

    out = pl.pallas_call(
        _masked_linear_kernel,
        grid=grid,
        in_specs=[
            pl.BlockSpec((_BLOCK_I, s, n), lambda i: (i, 0, 0)),
            pl.BlockSpec((_BLOCK_I, n), lambda i: (i, 0)),
            pl.BlockSpec((_PACK * h, _PACK * s), lambda i: (0, 0)),
            pl.BlockSpec((_PACK * h, _PACK * h), lambda i: (0, 0)),
        ],
        out_specs=pl.BlockSpec((_BLOCK_I, h, n), lambda i: (i, 0, 0)),
        out_shape=jax.ShapeDtypeStruct((rows, h, n), jnp.float32),
        compiler_params=pltpu.CompilerParams(
            dimension_semantics=("parallel",),
        ),
    )(xt, m2, a_big, b_big)

    return jnp.transpose(out.reshape(b, n, h, n), (0, 1, 3, 2))


# speedup vs baseline: 18.1047x; 1.0176x over previous
import jax
import jax.numpy as jnp
from jax.experimental import pallas as pl
from jax.experimental.pallas import tpu as pltpu

_S = 16
_PACK = 8
_BLOCK_I = 256
_N_LANES = 512


def _masked_linear_kernel(x_ref, m_ref, a_ref, b_ref, o_ref):
    a = a_ref[...]
    bcol = b_ref[...][:, 0:1]
    mf = m_ref[...].astype(jnp.float32)
    x = x_ref[...]
    for k in range(_BLOCK_I // _PACK):
        xk = x[_PACK * k : _PACK * (k + 1)].reshape(_PACK * _S, _N_LANES)
        y = jnp.dot(a, xk, preferred_element_type=jnp.float32) + bcol
        me = jnp.repeat(mf[_PACK * k : _PACK * k + _PACK, :], _S, axis=0)
        o_ref[_PACK * k : _PACK * (k + 1)] = (y * me).reshape(
            _PACK, _S, _N_LANES
        )


@jax.jit
def kernel(stacks, mask, W, bias):
    b, n, _, s = stacks.shape
    h = W.shape[0]
    rows = b * n

    xt = jnp.transpose(stacks, (0, 1, 3, 2)).reshape(rows, s, n)
    m2 = mask.reshape(rows, n)

    a_big = jnp.kron(jnp.eye(_PACK, dtype=W.dtype), W)
    b_big = jnp.tile(jnp.tile(bias, _PACK)[:, None], (1, _PACK * h))

    grid = (rows // _BLOCK_I,)

    out = pl.pallas_call(
        _masked_linear_kernel,
        grid=grid,
        in_specs=[
            pl.BlockSpec((_BLOCK_I, s, n), lambda i: (i, 0, 0)),
            pl.BlockSpec((_BLOCK_I, n), lambda i: (i, 0)),
            pl.BlockSpec((_PACK * h, _PACK * s), lambda i: (0, 0)),
            pl.BlockSpec((_PACK * h, _PACK * h), lambda i: (0, 0)),
        ],
        out_specs=pl.BlockSpec((_BLOCK_I, h, n), lambda i: (i, 0, 0)),
        out_shape=jax.ShapeDtypeStruct((rows, h, n), jnp.float32),
        compiler_params=pltpu.CompilerParams(
            dimension_semantics=("parallel",),
        ),
    )(xt, m2, a_big, b_big)

    return jnp.transpose(out.reshape(b, n, h, n), (0, 1, 3, 2))
